# Initial kernel scaffold; baseline (speedup 1.0000x reference)
import jax
import jax.numpy as jnp
from jax.experimental import pallas as pl


def kernel(x_category, x_item, item_avilability, category_coef, item_coef, lambdas):
    raise NotImplementedError("write your pallas kernel here")


# speedup vs baseline: 1.0806x; 1.0806x over previous
import functools

import jax
import jax.numpy as jnp
import numpy as np
from jax.experimental import pallas as pl
from jax.experimental.pallas import tpu as pltpu

NUM_CATEGORIES = 32
ITEMS_PER_CAT = 16
NUM_ITEMS = NUM_CATEGORIES * ITEMS_PER_CAT
CAT_FEAT = 128
ITEM_FEAT = 128
T_TILE = 32

_LOG_IPC = float(np.log(ITEMS_PER_CAT))


def _nested_logit_kernel(x_cat_ref, x_item_ref, avail_ref, cat_coef_ref,
                         item_coef_ref, lam_ref, inv_lam_ref, out_ref):
    xc = x_cat_ref[...]
    W = jnp.sum(xc * cat_coef_ref[...][None, :, :], axis=-1)

    xi = x_item_ref[...]
    Y = jnp.sum(xi * item_coef_ref[...][None, :, :], axis=-1)
    Y = jnp.where(avail_ref[...] > 0, Y, jnp.float32(-1e20))

    lam = lam_ref[...]
    inv_lam = inv_lam_ref[...]

    Y3 = Y.reshape(T_TILE, NUM_CATEGORIES, ITEMS_PER_CAT)
    Y3 = Y3 * inv_lam[:, :, None]

    seg_max = jnp.max(Y3, axis=-1)
    e = jnp.exp(Y3 - seg_max[:, :, None])
    seg_sum = jnp.sum(e, axis=-1)
    inclusive = seg_max + jnp.log(seg_sum)

    logit = W + lam * inclusive
    m = jnp.max(logit, axis=-1, keepdims=True)
    lse = m + jnp.log(jnp.sum(jnp.exp(logit - m), axis=-1, keepdims=True))
    lse = lse + jnp.float32(_LOG_IPC)
    logP_cat = logit - lse

    out3 = Y3 + (logP_cat - inclusive)[:, :, None]
    out_ref[...] = out3.reshape(T_TILE, NUM_ITEMS)


def kernel(x_category, x_item, item_avilability, category_coef, item_coef,
           lambdas):
    T = x_item.shape[0]
    grid = (T // T_TILE,)
    avail = item_avilability.astype(jnp.float32)
    lam = lambdas.reshape(1, NUM_CATEGORIES)
    inv_lam = (1.0 / lambdas).reshape(1, NUM_CATEGORIES)

    return pl.pallas_call(
        _nested_logit_kernel,
        grid=grid,
        in_specs=[
            pl.BlockSpec((T_TILE, NUM_CATEGORIES, CAT_FEAT),
                         lambda i: (i, 0, 0)),
            pl.BlockSpec((T_TILE, NUM_ITEMS, ITEM_FEAT),
                         lambda i: (i, 0, 0)),
            pl.BlockSpec((T_TILE, NUM_ITEMS), lambda i: (i, 0)),
            pl.BlockSpec((NUM_CATEGORIES, CAT_FEAT), lambda i: (0, 0)),
            pl.BlockSpec((NUM_ITEMS, ITEM_FEAT), lambda i: (0, 0)),
            pl.BlockSpec((1, NUM_CATEGORIES), lambda i: (0, 0)),
            pl.BlockSpec((1, NUM_CATEGORIES), lambda i: (0, 0)),
        ],
        out_specs=pl.BlockSpec((T_TILE, NUM_ITEMS), lambda i: (i, 0)),
        out_shape=jax.ShapeDtypeStruct((T, NUM_ITEMS), jnp.float32),
        compiler_params=pltpu.CompilerParams(
            dimension_semantics=("arbitrary",),
        ),
    )(x_category, x_item, avail, category_coef, item_coef, lam, inv_lam)
